# baseline (speedup 1.0000x reference)
import functools

import jax
import jax.numpy as jnp
from jax.experimental import pallas as pl
from jax.experimental.pallas import tpu as pltpu

_N = 4096
_BLK = 128
_GRID = _N // _BLK
_KTOP = 20
_MAXD = 0.01
_MAXD2 = 9.999999747378752e-05
_BASE_INV_LS = 1.0 / 0.006
_POWERS = (2, 6, 3)


def _enc_kernel(x_ref, mu_ref, sig_ref, w1_ref, b1_ref, w2_ref, b2_ref,
                w3_ref, b3_ref, wsb_ref, bsb_ref, wk_ref,
                h_out, heads_out, hw_out):
    xn = (x_ref[...] - mu_ref[...]) / (sig_ref[...] + 1e-06)
    h = jax.lax.dot_general(xn, w1_ref[...], (((1,), (0,)), ((), ())),
                            preferred_element_type=jnp.float32,
                            precision=jax.lax.Precision.HIGHEST)
    h = jnp.maximum(h + b1_ref[...], 0.0)
    h = jax.lax.dot_general(h, w2_ref[...], (((1,), (0,)), ((), ())),
                            preferred_element_type=jnp.float32,
                            precision=jax.lax.Precision.HIGHEST)
    h = jnp.maximum(h + b2_ref[...], 0.0)
    hh = jax.lax.dot_general(h, w3_ref[...], (((1,), (0,)), ((), ())),
                             preferred_element_type=jnp.float32,
                             precision=jax.lax.Precision.HIGHEST)
    hh = hh + b3_ref[...]
    h_out[...] = hh
    heads = jax.lax.dot_general(hh, wsb_ref[...], (((1,), (0,)), ((), ())),
                                preferred_element_type=jnp.float32,
                                precision=jax.lax.Precision.HIGHEST)
    heads_out[...] = heads + bsb_ref[...]
    for k in range(3):
        hw_out[k] = 0.25 * jax.lax.dot_general(
            hh, wk_ref[k], (((1,), (0,)), ((), ())),
            preferred_element_type=jnp.float32,
            precision=jax.lax.Precision.HIGHEST)


def _adj_kernel(hw_ref, ht_ref, cb_ref, ct_ref, out_ref):
    i = pl.program_id(0)
    cxb = cb_ref[:, 0:1]
    cyb = cb_ref[:, 1:2]
    cxa = ct_ref[0:1, :]
    cya = ct_ref[1:2, :]
    dx = cxb - cxa
    dy = cyb - cya
    r2 = dx * dx + dy * dy
    maskf = (r2 <= _MAXD2).astype(jnp.float32)
    d = r2 * jax.lax.rsqrt(r2 + 1e-30)
    em = jnp.exp(d * (-_BASE_INV_LS)) * maskf
    e2 = em * em
    e3 = e2 * em
    adist = (e2, e3 * e3, e3)
    cnt = jnp.sum(maskf, axis=1, keepdims=True)
    need_topk = jnp.max(cnt) > (_KTOP + 0.5)
    rloc = jax.lax.broadcasted_iota(jnp.int32, (_BLK, _BLK), 0)
    cloc = jax.lax.broadcasted_iota(jnp.int32, (_BLK, _BLK), 1)
    eye = rloc == cloc
    ht = ht_ref[...]
    hw_all = hw_ref[...].reshape(3 * _BLK, 64)
    s_all = jax.lax.dot_general(hw_all, ht, (((1,), (0,)), ((), ())),
                                preferred_element_type=jnp.float32)
    col0 = i * _BLK
    for k in range(3):
        s = jnp.tanh(s_all[k * _BLK:(k + 1) * _BLK, :])
        p = adist[k] * jnp.log1p(jnp.exp(s))
        out_ref[k] = p
        tile = out_ref[k, :, pl.ds(col0, _BLK)]
        out_ref[k, :, pl.ds(col0, _BLK)] = jnp.where(
            eye, jnp.float32(0.6931471805599453), tile)

        @pl.when(need_topk)
        def _(k=k):
            p_fix = out_ref[k]
            pw = p_fix
            m = None
            for _ in range(_KTOP):
                m = jnp.max(pw, axis=1, keepdims=True)
                pw = jnp.where(pw >= m, -1.0, pw)
            thr = jnp.maximum(m, 0.0)
            out_ref[k] = jnp.where(p_fix >= thr, p_fix, 0.0)


@jax.jit
def kernel(X_g, coords_g, mu, sigma, W1, b1, W2, b2, W3, b3, Ws, bs, Wk, Wb, bb):
    f32 = jnp.float32
    wsb = jnp.zeros((64, 128), f32).at[:, 0:3].set(Ws).at[:, 64:67].set(Wb)
    bsb = jnp.zeros((1, 128), f32).at[0, 0:3].set(bs).at[0, 64:67].set(bb)
    H, heads, HW = pl.pallas_call(
        _enc_kernel,
        out_shape=[
            jax.ShapeDtypeStruct((_N, 64), f32),
            jax.ShapeDtypeStruct((_N, 128), f32),
            jax.ShapeDtypeStruct((3, _N, 64), f32),
        ],
    )(X_g, mu.reshape(1, -1), sigma.reshape(1, -1), W1, b1.reshape(1, -1),
      W2, b2.reshape(1, -1), W3, b3.reshape(1, -1), wsb, bsb, Wk)
    w_self = heads[:, 0:3]
    baseline = heads[:, 64:67]

    A = pl.pallas_call(
        _adj_kernel,
        grid=(_GRID,),
        in_specs=[
            pl.BlockSpec((3, _BLK, 64), lambda i: (0, i, 0)),
            pl.BlockSpec((64, _N), lambda i: (0, 0)),
            pl.BlockSpec((_BLK, 2), lambda i: (i, 0)),
            pl.BlockSpec((2, _N), lambda i: (0, 0)),
        ],
        out_specs=pl.BlockSpec((3, _BLK, _N), lambda i: (0, i, 0)),
        out_shape=jax.ShapeDtypeStruct((3, _N, _N), f32),
        compiler_params=pltpu.CompilerParams(
            dimension_semantics=("parallel",)),
    )(HW.astype(jnp.bfloat16), H.T.astype(jnp.bfloat16),
      coords_g, coords_g.T)
    return (H, w_self, A, baseline)


# speedup vs baseline: 1.0950x; 1.0950x over previous
import functools

import jax
import jax.numpy as jnp
from jax.experimental import pallas as pl
from jax.experimental.pallas import tpu as pltpu

_N = 4096
_BLK = 128
_GRID = _N // _BLK
_KTOP = 20
_MAXD = 0.01
_MAXD2 = 9.999999747378752e-05
_BASE_INV_LS = 1.0 / 0.006
_POWERS = (2, 6, 3)


def _enc_kernel(x_ref, mu_ref, sig_ref, w1_ref, b1_ref, w2_ref, b2_ref,
                w3_ref, b3_ref, wsb_ref, bsb_ref, wk_ref,
                h_out, heads_out, hw_out):
    xn = (x_ref[...] - mu_ref[...]) / (sig_ref[...] + 1e-06)
    h = jax.lax.dot_general(xn, w1_ref[...], (((1,), (0,)), ((), ())),
                            preferred_element_type=jnp.float32)
    h = jnp.maximum(h + b1_ref[...], 0.0)
    h = jax.lax.dot_general(h, w2_ref[...], (((1,), (0,)), ((), ())),
                            preferred_element_type=jnp.float32)
    h = jnp.maximum(h + b2_ref[...], 0.0)
    hh = jax.lax.dot_general(h, w3_ref[...], (((1,), (0,)), ((), ())),
                             preferred_element_type=jnp.float32)
    hh = hh + b3_ref[...]
    h_out[...] = hh
    heads = jax.lax.dot_general(hh, wsb_ref[...], (((1,), (0,)), ((), ())),
                                preferred_element_type=jnp.float32)
    heads_out[...] = heads + bsb_ref[...]
    for k in range(3):
        hw_out[k] = 0.25 * jax.lax.dot_general(
            hh, wk_ref[k], (((1,), (0,)), ((), ())),
            preferred_element_type=jnp.float32)


def _adj_kernel(hw_ref, ht_ref, cb_ref, ct_ref, out_ref):
    i = pl.program_id(0)
    cxb = cb_ref[:, 0:1]
    cyb = cb_ref[:, 1:2]
    cxa = ct_ref[0:1, :]
    cya = ct_ref[1:2, :]
    dx = cxb - cxa
    dy = cyb - cya
    r2 = dx * dx + dy * dy
    maskf = (r2 <= _MAXD2).astype(jnp.float32)
    d = r2 * jax.lax.rsqrt(r2 + 1e-30)
    em = jnp.exp(d * (-_BASE_INV_LS)) * maskf
    e2 = em * em
    e3 = e2 * em
    adist = (e2, e3 * e3, e3)
    cnt = jnp.sum(maskf, axis=1, keepdims=True)
    need_topk = jnp.max(cnt) > (_KTOP + 0.5)
    rloc = jax.lax.broadcasted_iota(jnp.int32, (_BLK, _BLK), 0)
    cloc = jax.lax.broadcasted_iota(jnp.int32, (_BLK, _BLK), 1)
    eye = rloc == cloc
    ht = ht_ref[...]
    hw_all = hw_ref[...].reshape(3 * _BLK, 64)
    s_all = jax.lax.dot_general(hw_all, ht, (((1,), (0,)), ((), ())),
                                preferred_element_type=jnp.float32)
    col0 = i * _BLK
    for k in range(3):
        s = jnp.tanh(s_all[k * _BLK:(k + 1) * _BLK, :])
        p = adist[k] * jnp.log1p(jnp.exp(s))
        out_ref[k] = p
        tile = out_ref[k, :, pl.ds(col0, _BLK)]
        out_ref[k, :, pl.ds(col0, _BLK)] = jnp.where(
            eye, jnp.float32(0.6931471805599453), tile)

        @pl.when(need_topk)
        def _(k=k):
            p_fix = out_ref[k]
            pw = p_fix
            m = None
            for _ in range(_KTOP):
                m = jnp.max(pw, axis=1, keepdims=True)
                pw = jnp.where(pw >= m, -1.0, pw)
            thr = jnp.maximum(m, 0.0)
            out_ref[k] = jnp.where(p_fix >= thr, p_fix, 0.0)


@jax.jit
def kernel(X_g, coords_g, mu, sigma, W1, b1, W2, b2, W3, b3, Ws, bs, Wk, Wb, bb):
    f32 = jnp.float32
    wsb = jnp.zeros((64, 128), f32).at[:, 0:3].set(Ws).at[:, 64:67].set(Wb)
    bsb = jnp.zeros((1, 128), f32).at[0, 0:3].set(bs).at[0, 64:67].set(bb)
    H, heads, HW = pl.pallas_call(
        _enc_kernel,
        out_shape=[
            jax.ShapeDtypeStruct((_N, 64), f32),
            jax.ShapeDtypeStruct((_N, 128), f32),
            jax.ShapeDtypeStruct((3, _N, 64), f32),
        ],
    )(X_g, mu.reshape(1, -1), sigma.reshape(1, -1), W1, b1.reshape(1, -1),
      W2, b2.reshape(1, -1), W3, b3.reshape(1, -1), wsb, bsb, Wk)
    w_self = heads[:, 0:3]
    baseline = heads[:, 64:67]

    A = pl.pallas_call(
        _adj_kernel,
        grid=(_GRID,),
        in_specs=[
            pl.BlockSpec((3, _BLK, 64), lambda i: (0, i, 0)),
            pl.BlockSpec((64, _N), lambda i: (0, 0)),
            pl.BlockSpec((_BLK, 2), lambda i: (i, 0)),
            pl.BlockSpec((2, _N), lambda i: (0, 0)),
        ],
        out_specs=pl.BlockSpec((3, _BLK, _N), lambda i: (0, i, 0)),
        out_shape=jax.ShapeDtypeStruct((3, _N, _N), f32),
        compiler_params=pltpu.CompilerParams(
            dimension_semantics=("parallel",)),
    )(HW.astype(jnp.bfloat16), H.T.astype(jnp.bfloat16),
      coords_g, coords_g.T)
    return (H, w_self, A, baseline)


# speedup vs baseline: 1.0979x; 1.0027x over previous
import jax
import jax.numpy as jnp
from jax.experimental import pallas as pl
from jax.experimental.pallas import tpu as pltpu

_N = 4096
_BLK = 128
_GRID = _N // _BLK
_KTOP = 20
_MAXD2 = 9.999999747378752e-05
_BASE_INV_LS = 1.0 / 0.006


def _enc_kernel(x_ref, mu_ref, sig_ref, w1_ref, b1_ref, w2_ref, b2_ref,
                w3_ref, b3_ref, wsb_ref, bsb_ref, wk_ref,
                h_out, heads_out, hw_out):
    xn = (x_ref[...] - mu_ref[...]) / (sig_ref[...] + 1e-06)
    h = jax.lax.dot_general(xn, w1_ref[...], (((1,), (0,)), ((), ())),
                            preferred_element_type=jnp.float32)
    h = jnp.maximum(h + b1_ref[...], 0.0)
    h = jax.lax.dot_general(h, w2_ref[...], (((1,), (0,)), ((), ())),
                            preferred_element_type=jnp.float32)
    h = jnp.maximum(h + b2_ref[...], 0.0)
    hh = jax.lax.dot_general(h, w3_ref[...], (((1,), (0,)), ((), ())),
                             preferred_element_type=jnp.float32)
    hh = hh + b3_ref[...]
    h_out[...] = hh
    heads = jax.lax.dot_general(hh, wsb_ref[...], (((1,), (0,)), ((), ())),
                                preferred_element_type=jnp.float32)
    heads_out[...] = heads + bsb_ref[...]
    for k in range(3):
        hw_out[k] = 0.25 * jax.lax.dot_general(
            hh, wk_ref[k], (((1,), (0,)), ((), ())),
            preferred_element_type=jnp.float32)


def _adj_kernel(hw_ref, ht_ref, cb_ref, ct_ref, out_ref):
    i = pl.program_id(0)
    cxb = cb_ref[:, 0:1]
    cyb = cb_ref[:, 1:2]
    cxa = ct_ref[0:1, :]
    cya = ct_ref[1:2, :]
    dx = cxb - cxa
    dy = cyb - cya
    r2 = dx * dx + dy * dy
    maskf = (r2 <= _MAXD2).astype(jnp.float32)
    d = r2 * jax.lax.rsqrt(r2 + 1e-30)
    em = jnp.exp(d * (-_BASE_INV_LS)) * maskf
    e2 = em * em
    e3 = e2 * em
    adist = (e2, e3 * e3, e3)
    cnt = jnp.sum(maskf, axis=1, keepdims=True)
    need_topk = jnp.max(cnt) > (_KTOP + 0.5)
    rloc = jax.lax.broadcasted_iota(jnp.int32, (_BLK, _BLK), 0)
    cloc = jax.lax.broadcasted_iota(jnp.int32, (_BLK, _BLK), 1)
    eye = rloc == cloc
    ht = ht_ref[...]
    hw_all = hw_ref[...].reshape(3 * _BLK, 64)
    s_all = jax.lax.dot_general(hw_all, ht, (((1,), (0,)), ((), ())),
                                preferred_element_type=jnp.float32)
    col0 = i * _BLK
    for k in range(3):
        s = jnp.tanh(s_all[k * _BLK:(k + 1) * _BLK, :])
        p = adist[k] * jnp.log1p(jnp.exp(s))
        out_ref[k] = p
        tile = out_ref[k, :, pl.ds(col0, _BLK)]
        out_ref[k, :, pl.ds(col0, _BLK)] = jnp.where(
            eye, jnp.float32(0.6931471805599453), tile)

        @pl.when(need_topk)
        def _(k=k):
            p_fix = out_ref[k]
            pw = p_fix
            m = None
            for _ in range(_KTOP):
                m = jnp.max(pw, axis=1, keepdims=True)
                pw = jnp.where(pw >= m, -1.0, pw)
            thr = jnp.maximum(m, 0.0)
            out_ref[k] = jnp.where(p_fix >= thr, p_fix, 0.0)


@jax.jit
def kernel(X_g, coords_g, mu, sigma, W1, b1, W2, b2, W3, b3, Ws, bs, Wk, Wb, bb):
    f32 = jnp.float32
    wsb = jnp.zeros((64, 128), f32).at[:, 0:3].set(Ws).at[:, 64:67].set(Wb)
    bsb = jnp.zeros((1, 128), f32).at[0, 0:3].set(bs).at[0, 64:67].set(bb)
    H, heads, HW = pl.pallas_call(
        _enc_kernel,
        out_shape=[
            jax.ShapeDtypeStruct((_N, 64), f32),
            jax.ShapeDtypeStruct((_N, 128), f32),
            jax.ShapeDtypeStruct((3, _N, 64), f32),
        ],
    )(X_g, mu.reshape(1, -1), sigma.reshape(1, -1), W1, b1.reshape(1, -1),
      W2, b2.reshape(1, -1), W3, b3.reshape(1, -1), wsb, bsb, Wk)
    w_self = heads[:, 0:3]
    baseline = heads[:, 64:67]

    A = pl.pallas_call(
        _adj_kernel,
        grid=(_GRID,),
        in_specs=[
            pl.BlockSpec((3, _BLK, 64), lambda i: (0, i, 0)),
            pl.BlockSpec((64, _N), lambda i: (0, 0)),
            pl.BlockSpec((_BLK, 2), lambda i: (i, 0)),
            pl.BlockSpec((2, _N), lambda i: (0, 0)),
        ],
        out_specs=pl.BlockSpec((3, _BLK, _N), lambda i: (0, i, 0)),
        out_shape=jax.ShapeDtypeStruct((3, _N, _N), f32),
        compiler_params=pltpu.CompilerParams(
            dimension_semantics=("parallel",)),
    )(HW.astype(jnp.bfloat16), H.T.astype(jnp.bfloat16),
      coords_g, coords_g.T)
    return (H, w_self, A, baseline)
